# Initial kernel scaffold; baseline (speedup 1.0000x reference)
import jax
import jax.numpy as jnp
from jax.experimental import pallas as pl


def kernel(x_s, edge_index_s, x_t, edge_index_t, W_l, att_src_l, att_dst_l, bias_l, W_r, att_src_r, att_dst_r, bias_r, W1, b1, W2, b2, W3, b3):
    raise NotImplementedError("write your pallas kernel here")


# speedup vs baseline: 87.9531x; 87.9531x over previous
import functools

import jax
import jax.numpy as jnp
from jax import lax
from jax.experimental import pallas as pl
from jax.experimental.pallas import tpu as pltpu
from jax.experimental.pallas import tpu_sc as plsc

LANES = 16
CHW = 128
TILES = 16


def _logits_body(x_s_ref, x_t_ref, w_l_ref, w_r_ref, att_l_ref, att_r_ref,
                 out_ref):
    v_l = lax.dot_general(att_l_ref[...], w_l_ref[...],
                          (((1,), (0,)), ((), ())), precision=lax.Precision.HIGHEST)
    v_r = lax.dot_general(att_r_ref[...], w_r_ref[...],
                          (((1,), (0,)), ((), ())), precision=lax.Precision.HIGHEST)
    a_s = lax.dot_general(v_l, x_s_ref[...], (((1,), (1,)), ((), ())), precision=lax.Precision.HIGHEST)
    a_t = lax.dot_general(v_r, x_t_ref[...], (((1,), (1,)), ((), ())), precision=lax.Precision.HIGHEST)
    out_ref[...] = jnp.concatenate([a_s, a_t], axis=0)


def _logits(x_s_pad, x_t_pad, w_l, w_r, att_l, att_r, n_pad):
    return pl.pallas_call(
        _logits_body,
        out_shape=jax.ShapeDtypeStruct((4, n_pad), jnp.float32),
    )(x_s_pad, x_t_pad, w_l, w_r, att_l, att_r)


def _softmax_body(n_pad, ch, src_hbm, dst_hbm, a_hbm, w_hbm,
                  src_v, dst_v, ex_v, av_v, den_v, den_sh, w_sh):
    c = lax.axis_index("c")
    s = lax.axis_index("s")

    @pl.when(s == 0)
    def _():
        def zero(i, carry):
            den_v[pl.ds(i * LANES, LANES)] = jnp.zeros((LANES,), jnp.float32)
            return carry
        lax.fori_loop(0, n_pad // LANES, zero, 0)
        pltpu.sync_copy(den_v, den_sh)
        pltpu.sync_copy(den_v, w_sh)

    base = s * ch
    pltpu.sync_copy(src_hbm.at[c, pl.ds(base, ch)], src_v)
    pltpu.sync_copy(dst_hbm.at[c, pl.ds(base, ch)], dst_v)
    pltpu.sync_copy(a_hbm.at[pl.ds(2 * c * n_pad, 2 * n_pad)], av_v)
    plsc.subcore_barrier()

    def p1(j, carry):
        for k in range(CHW // LANES):
            ii = pl.ds(k * LANES, LANES)
            al = (plsc.load_gather(av_v, [src_v[j, ii]])
                  + plsc.load_gather(av_v, [dst_v[j, ii] + n_pad]))
            ex_v[j, ii] = jnp.exp(jnp.maximum(al, al * 0.2))
        return carry
    lax.fori_loop(0, ch, p1, 0)

    def p1s(j, carry):
        pltpu.sync_copy(ex_v.at[j], den_sh.at[dst_v.at[j]], add=True)
        return carry
    lax.fori_loop(0, ch, p1s, 0)
    plsc.subcore_barrier()

    pltpu.sync_copy(den_sh, den_v)

    def p2(j, carry):
        for k in range(CHW // LANES):
            ii = pl.ds(k * LANES, LANES)
            dv = plsc.load_gather(den_v, [dst_v[j, ii]])
            ex_v[j, ii] = ex_v[j, ii] / dv
        return carry
    lax.fori_loop(0, ch, p2, 0)

    def p2s(j, carry):
        pltpu.sync_copy(ex_v.at[j], w_sh.at[src_v.at[j]], add=True)
        return carry
    lax.fori_loop(0, ch, p2s, 0)
    plsc.subcore_barrier()

    @pl.when(s == 0)
    def _():
        pltpu.sync_copy(w_sh, w_hbm.at[pl.ds(c * n_pad, n_pad)])


def _edge_softmax(src2, dst2, a_flat, n_pad, ch):
    mesh = plsc.VectorSubcoreMesh(core_axis_name="c", subcore_axis_name="s")
    return pl.kernel(
        functools.partial(_softmax_body, n_pad, ch),
        out_type=jax.ShapeDtypeStruct((2 * n_pad,), jnp.float32),
        mesh=mesh,
        compiler_params=pltpu.CompilerParams(needs_layout_passes=False),
        scratch_types=[
            pltpu.VMEM((ch, CHW), jnp.int32),
            pltpu.VMEM((ch, CHW), jnp.int32),
            pltpu.VMEM((ch, CHW), jnp.float32),
            pltpu.VMEM((2 * n_pad,), jnp.float32),
            pltpu.VMEM((n_pad,), jnp.float32),
            pltpu.VMEM_SHARED((n_pad,), jnp.float32),
            pltpu.VMEM_SHARED((n_pad,), jnp.float32),
        ],
    )(src2, dst2, a_flat)


def _head_body(n_nodes, w2_ref, x_s_ref, x_t_ref, w_l_ref, w_r_ref,
               b_l_ref, b_r_ref, w1_ref, b1_ref, w2m_ref, b2_ref,
               w3_ref, b3_ref, out_ref):
    nn = jnp.float32(n_nodes)
    xw_s = lax.dot_general(w2_ref[0:1, :], x_s_ref[...],
                           (((1,), (0,)), ((), ())), precision=lax.Precision.HIGHEST)
    xw_t = lax.dot_general(w2_ref[1:2, :], x_t_ref[...],
                           (((1,), (0,)), ((), ())), precision=lax.Precision.HIGHEST)
    sum_a = lax.dot_general(xw_s, w_l_ref[...],
                            (((1,), (1,)), ((), ())), precision=lax.Precision.HIGHEST) + nn * b_l_ref[...][None, :]
    sum_b = lax.dot_general(xw_t, w_r_ref[...],
                            (((1,), (1,)), ((), ())), precision=lax.Precision.HIGHEST) + nn * b_r_ref[...][None, :]
    featc = jnp.concatenate([sum_a, sum_b], axis=1)
    h1 = lax.dot_general(w1_ref[...], featc,
                         (((1,), (1,)), ((), ())), precision=lax.Precision.HIGHEST)
    h1 = h1 + b1_ref[...][:, None]
    h2 = lax.dot_general(w2m_ref[...], h1,
                         (((1,), (0,)), ((), ())), precision=lax.Precision.HIGHEST)
    h2 = h2 + b2_ref[...][:, None]
    out = lax.dot_general(w3_ref[...], h2,
                          (((1,), (0,)), ((), ())), precision=lax.Precision.HIGHEST)
    out_ref[...] = out + b3_ref[...][:, None]


def _head(w2, x_s, x_t, w_l, w_r, b_l, b_r, w1, b1, w2m, b2, w3, b3):
    n_nodes = x_s.shape[0]
    return pl.pallas_call(
        functools.partial(_head_body, n_nodes),
        out_shape=jax.ShapeDtypeStruct((1, 1), jnp.float32),
    )(w2, x_s, x_t, w_l, w_r, b_l, b_r, w1, b1, w2m, b2, w3, b3)


def kernel(x_s, edge_index_s, x_t, edge_index_t, W_l, att_src_l, att_dst_l,
           bias_l, W_r, att_src_r, att_dst_r, bias_r, W1, b1, W2, b2, W3, b3):
    n, c = x_s.shape
    e = edge_index_s.shape[1]
    n_pad = ((n + 255) // 256) * 256
    et = e + n
    ch = (et + TILES * CHW - 1) // (TILES * CHW)
    ch = ((ch + 7) // 8) * 8
    e_pad = ch * CHW * TILES

    idt = edge_index_s.dtype
    loop = jnp.arange(n, dtype=idt)
    pad = jnp.full((e_pad - et,), n, dtype=idt)

    def edges(ei):
        src = jnp.concatenate([ei[0], loop, pad]).reshape(TILES * ch, CHW)
        dst = jnp.concatenate([ei[1], loop, pad]).reshape(TILES * ch, CHW)
        return src, dst

    src_s, dst_s = edges(edge_index_s)
    src_t, dst_t = edges(edge_index_t)
    src2 = jnp.stack([src_s, src_t])
    dst2 = jnp.stack([dst_s, dst_t])

    zpad = jnp.zeros((n_pad - n, c), jnp.float32)
    x_s_pad = jnp.concatenate([x_s, zpad], axis=0)
    x_t_pad = jnp.concatenate([x_t, zpad], axis=0)
    att_l = jnp.stack([att_src_l, att_dst_l])
    att_r = jnp.stack([att_src_r, att_dst_r])

    a4 = _logits(x_s_pad, x_t_pad, W_l, W_r, att_l, att_r, n_pad)
    w_flat = _edge_softmax(src2, dst2, a4.reshape(-1), n_pad, ch)
    w2 = w_flat.reshape(2, n_pad)[:, :n]
    out = _head(w2, x_s, x_t, W_l, W_r, bias_l, bias_r,
                W1, b1, W2, b2, W3, b3)
    return out.reshape(1)


# speedup vs baseline: 110.8562x; 1.2604x over previous
import functools

import jax
import jax.numpy as jnp
from jax import lax
from jax.experimental import pallas as pl
from jax.experimental.pallas import tpu as pltpu
from jax.experimental.pallas import tpu_sc as plsc

LANES = 16
CHW = 128
TILES = 16


def _logits_body(x_s_ref, x_t_ref, w_l_ref, w_r_ref, att_l_ref, att_r_ref,
                 out_ref):
    v_l = lax.dot_general(att_l_ref[...], w_l_ref[...],
                          (((1,), (0,)), ((), ())), precision=lax.Precision.HIGHEST)
    v_r = lax.dot_general(att_r_ref[...], w_r_ref[...],
                          (((1,), (0,)), ((), ())), precision=lax.Precision.HIGHEST)
    a_s = lax.dot_general(v_l, x_s_ref[...], (((1,), (1,)), ((), ())), precision=lax.Precision.HIGHEST)
    a_t = lax.dot_general(v_r, x_t_ref[...], (((1,), (1,)), ((), ())), precision=lax.Precision.HIGHEST)
    out_ref[...] = jnp.concatenate([a_s, a_t], axis=0)


def _logits(x_s_pad, x_t_pad, w_l, w_r, att_l, att_r, n_pad):
    return pl.pallas_call(
        _logits_body,
        out_shape=jax.ShapeDtypeStruct((4, n_pad), jnp.float32),
    )(x_s_pad, x_t_pad, w_l, w_r, att_l, att_r)


def _softmax_body(n_pad, ept, src_hbm, dst_hbm, a_hbm, w_hbm,
                  src_v, dst_v, ex_v, av_v, den_v, den_sh, w_sh):
    c = lax.axis_index("c")
    s = lax.axis_index("s")

    @pl.when(s == 0)
    def _():
        def zero(i, carry):
            den_v[pl.ds(i * LANES, LANES)] = jnp.zeros((LANES,), jnp.float32)
            return carry
        lax.fori_loop(0, n_pad // LANES, zero, 0)
        pltpu.sync_copy(den_v, den_sh)
        pltpu.sync_copy(den_v, w_sh)

    base = c * (TILES * ept) + s * ept
    pltpu.sync_copy(src_hbm.at[pl.ds(base, ept)], src_v)
    pltpu.sync_copy(dst_hbm.at[pl.ds(base, ept)], dst_v)
    pltpu.sync_copy(a_hbm.at[pl.ds(2 * c * n_pad, 2 * n_pad)], av_v)
    plsc.subcore_barrier()

    def p1(j, carry):
        ii = pl.ds(j * LANES, LANES)
        al = (plsc.load_gather(av_v, [src_v[ii]])
              + plsc.load_gather(av_v, [dst_v[ii] + n_pad]))
        ex_v[ii] = jnp.exp(jnp.maximum(al, al * 0.2))
        return carry
    lax.fori_loop(0, ept // LANES, p1, 0)

    pltpu.sync_copy(ex_v, den_sh.at[dst_v], add=True)
    plsc.subcore_barrier()

    pltpu.sync_copy(den_sh, den_v)

    def p2(j, carry):
        ii = pl.ds(j * LANES, LANES)
        dv = plsc.load_gather(den_v, [dst_v[ii]])
        ex_v[ii] = ex_v[ii] / dv
        return carry
    lax.fori_loop(0, ept // LANES, p2, 0)

    pltpu.sync_copy(ex_v, w_sh.at[src_v], add=True)
    plsc.subcore_barrier()

    @pl.when(s == 0)
    def _():
        pltpu.sync_copy(w_sh, w_hbm.at[pl.ds(c * n_pad, n_pad)])


def _edge_softmax(src2, dst2, a_flat, n_pad, ept):
    mesh = plsc.VectorSubcoreMesh(core_axis_name="c", subcore_axis_name="s")
    return pl.kernel(
        functools.partial(_softmax_body, n_pad, ept),
        out_type=jax.ShapeDtypeStruct((2 * n_pad,), jnp.float32),
        mesh=mesh,
        compiler_params=pltpu.CompilerParams(needs_layout_passes=False),
        scratch_types=[
            pltpu.VMEM((ept,), jnp.int32),
            pltpu.VMEM((ept,), jnp.int32),
            pltpu.VMEM((ept,), jnp.float32),
            pltpu.VMEM((2 * n_pad,), jnp.float32),
            pltpu.VMEM((n_pad,), jnp.float32),
            pltpu.VMEM_SHARED((n_pad,), jnp.float32),
            pltpu.VMEM_SHARED((n_pad,), jnp.float32),
        ],
    )(src2, dst2, a_flat)


def _head_body(n_nodes, w2_ref, x_s_ref, x_t_ref, w_l_ref, w_r_ref,
               b_l_ref, b_r_ref, w1_ref, b1_ref, w2m_ref, b2_ref,
               w3_ref, b3_ref, out_ref):
    nn = jnp.float32(n_nodes)
    xw_s = lax.dot_general(w2_ref[0:1, :], x_s_ref[...],
                           (((1,), (0,)), ((), ())), precision=lax.Precision.HIGHEST)
    xw_t = lax.dot_general(w2_ref[1:2, :], x_t_ref[...],
                           (((1,), (0,)), ((), ())), precision=lax.Precision.HIGHEST)
    sum_a = lax.dot_general(xw_s, w_l_ref[...],
                            (((1,), (1,)), ((), ())), precision=lax.Precision.HIGHEST) + nn * b_l_ref[...][None, :]
    sum_b = lax.dot_general(xw_t, w_r_ref[...],
                            (((1,), (1,)), ((), ())), precision=lax.Precision.HIGHEST) + nn * b_r_ref[...][None, :]
    featc = jnp.concatenate([sum_a, sum_b], axis=1)
    h1 = lax.dot_general(w1_ref[...], featc,
                         (((1,), (1,)), ((), ())), precision=lax.Precision.HIGHEST)
    h1 = h1 + b1_ref[...][:, None]
    h2 = lax.dot_general(w2m_ref[...], h1,
                         (((1,), (0,)), ((), ())), precision=lax.Precision.HIGHEST)
    h2 = h2 + b2_ref[...][:, None]
    out = lax.dot_general(w3_ref[...], h2,
                          (((1,), (0,)), ((), ())), precision=lax.Precision.HIGHEST)
    out_ref[...] = out + b3_ref[...][:, None]


def _head(w2, x_s, x_t, w_l, w_r, b_l, b_r, w1, b1, w2m, b2, w3, b3):
    n_nodes = x_s.shape[0]
    return pl.pallas_call(
        functools.partial(_head_body, n_nodes),
        out_shape=jax.ShapeDtypeStruct((1, 1), jnp.float32),
    )(w2, x_s, x_t, w_l, w_r, b_l, b_r, w1, b1, w2m, b2, w3, b3)


def kernel(x_s, edge_index_s, x_t, edge_index_t, W_l, att_src_l, att_dst_l,
           bias_l, W_r, att_src_r, att_dst_r, bias_r, W1, b1, W2, b2, W3, b3):
    n, c = x_s.shape
    e = edge_index_s.shape[1]
    n_pad = ((n + 255) // 256) * 256
    et = e + n
    ept = (et + TILES - 1) // TILES
    ept = ((ept + 127) // 128) * 128
    e_pad = ept * TILES

    idt = edge_index_s.dtype
    loop = jnp.arange(n, dtype=idt)
    pad = jnp.full((e_pad - et,), n, dtype=idt)

    def edges(ei):
        src = jnp.concatenate([ei[0], loop, pad])
        dst = jnp.concatenate([ei[1], loop, pad])
        return src, dst

    src_s, dst_s = edges(edge_index_s)
    src_t, dst_t = edges(edge_index_t)
    src2 = jnp.concatenate([src_s, src_t])
    dst2 = jnp.concatenate([dst_s, dst_t])

    zpad = jnp.zeros((n_pad - n, c), jnp.float32)
    x_s_pad = jnp.concatenate([x_s, zpad], axis=0)
    x_t_pad = jnp.concatenate([x_t, zpad], axis=0)
    att_l = jnp.stack([att_src_l, att_dst_l])
    att_r = jnp.stack([att_src_r, att_dst_r])

    a4 = _logits(x_s_pad, x_t_pad, W_l, W_r, att_l, att_r, n_pad)
    w_flat = _edge_softmax(src2, dst2, a4.reshape(-1), n_pad, ept)
    w2 = w_flat.reshape(2, n_pad)[:, :n]
    out = _head(w2, x_s, x_t, W_l, W_r, bias_l, bias_r,
                W1, b1, W2, b2, W3, b3)
    return out.reshape(1)
